# baseline (speedup 1.0000x reference)
import functools

import jax
import jax.numpy as jnp
from jax import lax
from jax.experimental import pallas as pl
from jax.experimental.pallas import tpu as pltpu
from jax.experimental.pallas import tpu_sc as plsc

_N_IMGS = 100000
_BATCH = 16384
_NW = 32
_PER_W = _BATCH // _NW
_CHUNK = 128
_NCHUNK = _PER_W // _CHUNK
_L = 16
_PHI_ROWS = _N_IMGS * 4 // _L
_T_ROWS = _N_IMGS * 3 // _L
_PHI_GROUPS = _PER_W * 4 // _L
_T_GROUPS = _PER_W * 3 // _L


def _build_gather_kernel():
    mesh = plsc.VectorSubcoreMesh(core_axis_name="c", subcore_axis_name="s")

    @functools.partial(
        pl.kernel,
        mesh=mesh,
        out_type=[
            jax.ShapeDtypeStruct((_BATCH * 4,), jnp.float32),
            jax.ShapeDtypeStruct((_BATCH * 3,), jnp.float32),
            jax.ShapeDtypeStruct((_L,), jnp.float32),
        ],
        scratch_types=[
            pltpu.VMEM((_PER_W,), jnp.int32),
            pltpu.VMEM((_PER_W,), jnp.int32),
            pltpu.VMEM((_PER_W,), jnp.int32),
            pltpu.VMEM((_PER_W,), jnp.int32),
            pltpu.VMEM((_PER_W, _L), jnp.float32),
            pltpu.VMEM((_PER_W, _L), jnp.float32),
            pltpu.VMEM((_PER_W, _L), jnp.float32),
            pltpu.VMEM((_PER_W * 4,), jnp.float32),
            pltpu.VMEM((_PER_W * 3,), jnp.float32),
            pltpu.VMEM((_L,), jnp.float32),
            pltpu.SemaphoreType.DMA,
            pltpu.SemaphoreType.DMA,
        ],
        compiler_params=pltpu.CompilerParams(
            use_tc_tiling_on_sc=False, needs_layout_passes=False),
    )
    def gather_k(phi_hbm, t_hbm, f_hbm, idx_hbm,
                 phi_out, t_out, fx_out,
                 idx_v, gp_v, ga_v, gb_v, phi_rows, ta_rows, tb_rows,
                 phi_x, t_x, f_v, sem_a, sem_b):
        wid = lax.axis_index("s") * 2 + lax.axis_index("c")
        base = wid * _PER_W

        pltpu.sync_copy(idx_hbm.at[pl.ds(base, _PER_W)], idx_v)

        for s in range(_PER_W // _L):
            v = idx_v[pl.ds(s * _L, _L)]
            gp_v[pl.ds(s * _L, _L)] = lax.shift_right_logical(v, 2)
            s3 = v * 3
            ga = lax.shift_right_logical(s3, 4)
            ga_v[pl.ds(s * _L, _L)] = ga
            gb_v[pl.ds(s * _L, _L)] = jnp.minimum(ga + 1, _T_ROWS - 1)

        copies = []
        for j in range(_NCHUNK):
            sl = pl.ds(j * _CHUNK, _CHUNK)
            copies.append(pltpu.async_copy(
                phi_hbm.at[gp_v.at[sl]], phi_rows.at[sl], sem_a))
            copies.append(pltpu.async_copy(
                t_hbm.at[ga_v.at[sl]], ta_rows.at[sl], sem_b))
            copies.append(pltpu.async_copy(
                t_hbm.at[gb_v.at[sl]], tb_rows.at[sl], sem_b))
        for c in copies:
            c.wait()

        lanes = lax.iota(jnp.int32, _L)

        def phi_body(k, carry):
            p = k * _L + lanes
            i = lax.shift_right_logical(p, 2)
            c = lax.bitwise_and(p, 3)
            idxl = plsc.load_gather(idx_v, [i])
            col = lax.shift_left(lax.bitwise_and(idxl, 3), 2) + c
            val = plsc.load_gather(phi_rows, [i, col])
            plsc.store_scatter(phi_x, [p], val)
            return carry

        lax.fori_loop(0, _PHI_GROUPS, phi_body, 0)

        def t_body(k, carry):
            p = k * _L + lanes
            i = p // 3
            c = p - i * 3
            idxl = plsc.load_gather(idx_v, [i])
            s0 = idxl * 3
            g = lax.shift_right_logical(s0, 4)
            col = s0 + c - lax.shift_left(g, 4)
            in_a = col < _L
            val_a = plsc.load_gather(ta_rows, [i, jnp.minimum(col, _L - 1)])
            val_b = plsc.load_gather(tb_rows, [i, jnp.maximum(col - _L, 0)])
            plsc.store_scatter(t_x, [p], jnp.where(in_a, val_a, val_b))
            return carry

        lax.fori_loop(0, _T_GROUPS, t_body, 0)

        pltpu.sync_copy(phi_x, phi_out.at[pl.ds(base * 4, _PER_W * 4)])
        pltpu.sync_copy(t_x, t_out.at[pl.ds(base * 3, _PER_W * 3)])

        @pl.when(wid == 0)
        def _():
            pltpu.sync_copy(f_hbm, f_v)
            val = f_v[...]
            f_v[...] = val * val * 1000.0
            pltpu.sync_copy(f_v, fx_out)

    return gather_k


_gather = _build_gather_kernel()


def kernel(phi, t, f, indices):
    idx = indices.astype(jnp.int32)
    phi_r = phi.reshape(_PHI_ROWS, _L)
    t_r = t.reshape(_T_ROWS, _L)
    f16 = jnp.broadcast_to(f.astype(jnp.float32), (_L,))
    phi_flat, t_flat, fx16 = _gather(phi_r, t_r, f16, idx)
    phi_sel = phi_flat.reshape(_BATCH, 4)
    t_sel = t_flat.reshape(_BATCH, 3)
    fx = fx16[:1]
    cx = jnp.asarray(500.0, jnp.float32)
    cy = jnp.asarray(500.0, jnp.float32)
    return (phi_sel, t_sel, fx, fx, cx, cy)


# speedup vs baseline: 4.7753x; 4.7753x over previous
import functools

import jax
import jax.numpy as jnp
from jax import lax
from jax.experimental import pallas as pl
from jax.experimental.pallas import tpu as pltpu
from jax.experimental.pallas import tpu_sc as plsc

_N_IMGS = 100000
_BATCH = 16384
_NW = 32
_PER_W = _BATCH // _NW
_CHUNK = 128
_NCHUNK = _PER_W // _CHUNK
_L = 16
_CSTRIDE = _N_IMGS // _L


def _build_gather_kernel():
    mesh = plsc.VectorSubcoreMesh(core_axis_name="c", subcore_axis_name="s")

    @functools.partial(
        pl.kernel,
        mesh=mesh,
        out_type=[
            jax.ShapeDtypeStruct((_BATCH * 4,), jnp.float32),
            jax.ShapeDtypeStruct((_BATCH * 3,), jnp.float32),
            jax.ShapeDtypeStruct((_L,), jnp.float32),
        ],
        scratch_types=[
            pltpu.VMEM((_PER_W,), jnp.int32),
            pltpu.VMEM((4 * _PER_W,), jnp.int32),
            pltpu.VMEM((4 * _PER_W, _L), jnp.float32),
            pltpu.VMEM((3 * _PER_W, _L), jnp.float32),
            pltpu.VMEM((4 * _PER_W,), jnp.float32),
            pltpu.VMEM((3 * _PER_W,), jnp.float32),
            pltpu.VMEM((_L,), jnp.float32),
            pltpu.SemaphoreType.DMA,
            pltpu.SemaphoreType.DMA,
        ],
        compiler_params=pltpu.CompilerParams(
            use_tc_tiling_on_sc=False, needs_layout_passes=False),
    )
    def gather_k(phi_hbm, t_hbm, f_hbm, idx_hbm,
                 phi_out, t_out, fx_out,
                 idx_v, gidx_v, phi_rows, t_rows, phi_x, t_x, f_v,
                 sem_a, sem_b):
        wid = lax.axis_index("s") * 2 + lax.axis_index("c")
        base = wid * _PER_W

        pltpu.sync_copy(idx_hbm.at[pl.ds(base, _PER_W)], idx_v)

        for s in range(_PER_W // _L):
            g0 = lax.shift_right_logical(idx_v[pl.ds(s * _L, _L)], 4)
            for c in range(4):
                gidx_v[pl.ds(c * _PER_W + s * _L, _L)] = g0 + (c * _CSTRIDE)

        copies = []
        for c in range(4):
            for j in range(_NCHUNK):
                sl = pl.ds(c * _PER_W + j * _CHUNK, _CHUNK)
                copies.append(pltpu.async_copy(
                    phi_hbm.at[gidx_v.at[sl]], phi_rows.at[sl], sem_a))
                if c < 3:
                    copies.append(pltpu.async_copy(
                        t_hbm.at[gidx_v.at[sl]], t_rows.at[sl], sem_b))
        for c in copies:
            c.wait()

        lanes = lax.iota(jnp.int32, _L)

        for s in range(_PER_W // _L):
            sl = pl.ds(s * _L, _L)
            col = lax.bitwise_and(idx_v[sl], _L - 1)
            row = s * _L + lanes
            for c in range(4):
                val = plsc.load_gather(phi_rows, [c * _PER_W + row, col])
                phi_x[pl.ds(c * _PER_W + s * _L, _L)] = val
            for c in range(3):
                val = plsc.load_gather(t_rows, [c * _PER_W + row, col])
                t_x[pl.ds(c * _PER_W + s * _L, _L)] = val

        for c in range(4):
            pltpu.sync_copy(phi_x.at[pl.ds(c * _PER_W, _PER_W)],
                            phi_out.at[pl.ds(c * _BATCH + base, _PER_W)])
        for c in range(3):
            pltpu.sync_copy(t_x.at[pl.ds(c * _PER_W, _PER_W)],
                            t_out.at[pl.ds(c * _BATCH + base, _PER_W)])

        @pl.when(wid == 0)
        def _():
            pltpu.sync_copy(f_hbm, f_v)
            val = f_v[...]
            f_v[...] = val * val * 1000.0
            pltpu.sync_copy(f_v, fx_out)

    return gather_k


_gather = _build_gather_kernel()


def kernel(phi, t, f, indices):
    idx = indices.astype(jnp.int32)
    phi_cm = phi.T.reshape(4 * _CSTRIDE, _L)
    t_cm = t.T.reshape(3 * _CSTRIDE, _L)
    f16 = jnp.broadcast_to(f.astype(jnp.float32), (_L,))
    phi_flat, t_flat, fx16 = _gather(phi_cm, t_cm, f16, idx)
    phi_sel = phi_flat.reshape(4, _BATCH).T
    t_sel = t_flat.reshape(3, _BATCH).T
    fx = fx16[:1]
    cx = jnp.asarray(500.0, jnp.float32)
    cy = jnp.asarray(500.0, jnp.float32)
    return (phi_sel, t_sel, fx, fx, cx, cy)


# speedup vs baseline: 4.8168x; 1.0087x over previous
import functools

import jax
import jax.numpy as jnp
from jax import lax
from jax.experimental import pallas as pl
from jax.experimental.pallas import tpu as pltpu
from jax.experimental.pallas import tpu_sc as plsc

_N_IMGS = 100000
_BATCH = 16384
_NW = 32
_PER_W = _BATCH // _NW
_L = 16
_CSTRIDE = _N_IMGS // _L


def _build_gather_kernel():
    mesh = plsc.VectorSubcoreMesh(core_axis_name="c", subcore_axis_name="s")

    @functools.partial(
        pl.kernel,
        mesh=mesh,
        out_type=[
            jax.ShapeDtypeStruct((_BATCH * 4,), jnp.float32),
            jax.ShapeDtypeStruct((_BATCH * 3,), jnp.float32),
            jax.ShapeDtypeStruct((_L,), jnp.float32),
        ],
        scratch_types=[
            pltpu.VMEM((_PER_W,), jnp.int32),
            pltpu.VMEM((_PER_W,), jnp.int32),
            pltpu.VMEM((4 * _PER_W, _L), jnp.float32),
            pltpu.VMEM((3 * _PER_W, _L), jnp.float32),
            pltpu.VMEM((4 * _PER_W,), jnp.float32),
            pltpu.VMEM((3 * _PER_W,), jnp.float32),
            pltpu.VMEM((_L,), jnp.float32),
            pltpu.SemaphoreType.DMA,
            pltpu.SemaphoreType.DMA,
        ],
        compiler_params=pltpu.CompilerParams(
            use_tc_tiling_on_sc=False, needs_layout_passes=False),
    )
    def gather_k(phi_hbm, t_hbm, f_hbm, idx_hbm,
                 phi_out, t_out, fx_out,
                 idx_v, gbase_v, phi_rows, t_rows, phi_x, t_x, f_v,
                 sem_a, sem_b):
        wid = lax.axis_index("s") * 2 + lax.axis_index("c")
        base = wid * _PER_W

        pltpu.sync_copy(idx_hbm.at[pl.ds(base, _PER_W)], idx_v)

        for s in range(_PER_W // _L):
            sl = pl.ds(s * _L, _L)
            gbase_v[sl] = lax.shift_right_logical(idx_v[sl], 4)

        copies = []
        for c in range(4):
            col = phi_hbm.at[pl.ds(c * _CSTRIDE, _CSTRIDE)]
            copies.append(pltpu.async_copy(
                col.at[gbase_v], phi_rows.at[pl.ds(c * _PER_W, _PER_W)],
                sem_a))
        for c in range(3):
            col = t_hbm.at[pl.ds(c * _CSTRIDE, _CSTRIDE)]
            copies.append(pltpu.async_copy(
                col.at[gbase_v], t_rows.at[pl.ds(c * _PER_W, _PER_W)],
                sem_b))
        for c in copies:
            c.wait()

        lanes = lax.iota(jnp.int32, _L)

        for s in range(_PER_W // _L):
            sl = pl.ds(s * _L, _L)
            col = lax.bitwise_and(idx_v[sl], _L - 1)
            row = s * _L + lanes
            for c in range(4):
                val = plsc.load_gather(phi_rows, [c * _PER_W + row, col])
                phi_x[pl.ds(c * _PER_W + s * _L, _L)] = val
            for c in range(3):
                val = plsc.load_gather(t_rows, [c * _PER_W + row, col])
                t_x[pl.ds(c * _PER_W + s * _L, _L)] = val

        for c in range(4):
            pltpu.sync_copy(phi_x.at[pl.ds(c * _PER_W, _PER_W)],
                            phi_out.at[pl.ds(c * _BATCH + base, _PER_W)])
        for c in range(3):
            pltpu.sync_copy(t_x.at[pl.ds(c * _PER_W, _PER_W)],
                            t_out.at[pl.ds(c * _BATCH + base, _PER_W)])

        @pl.when(wid == 0)
        def _():
            pltpu.sync_copy(f_hbm, f_v)
            val = f_v[...]
            f_v[...] = val * val * 1000.0
            pltpu.sync_copy(f_v, fx_out)

    return gather_k


_gather = _build_gather_kernel()


def kernel(phi, t, f, indices):
    idx = indices.astype(jnp.int32)
    phi_cm = phi.T.reshape(4 * _CSTRIDE, _L)
    t_cm = t.T.reshape(3 * _CSTRIDE, _L)
    f16 = jnp.broadcast_to(f.astype(jnp.float32), (_L,))
    phi_flat, t_flat, fx16 = _gather(phi_cm, t_cm, f16, idx)
    phi_sel = phi_flat.reshape(4, _BATCH).T
    t_sel = t_flat.reshape(3, _BATCH).T
    fx = fx16[:1]
    cx = jnp.asarray(500.0, jnp.float32)
    cy = jnp.asarray(500.0, jnp.float32)
    return (phi_sel, t_sel, fx, fx, cx, cy)
